# Initial kernel scaffold; baseline (speedup 1.0000x reference)
import jax
import jax.numpy as jnp
from jax.experimental import pallas as pl


def kernel(x, edge_index, W1_l, W1_r, b1, W2_l, W2_r, b2):
    raise NotImplementedError("write your pallas kernel here")


# speedup vs baseline: 4.2098x; 4.2098x over previous
import functools

import jax
import jax.numpy as jnp
from jax import lax
from jax.experimental import pallas as pl
from jax.experimental.pallas import tpu as pltpu
from jax.experimental.pallas import tpu_sc as plsc

_N = 10000
_D = 128
_NP = 10240
_E = 320000
_C = 128
_NTILES = 32
_NCHUNK = 79
_EPT = _NCHUNK * _C
_EP = _NTILES * _EPT
_RPT = _NP // 16


def _sc_agg_body(with_counts, *refs):
    if with_counts:
        (x_hbm, src_hbm, dst_hbm, out_hbm, cnt_hbm,
         acc_sh, cnt_sh, src_v, dst_v, rows_v, ones_v, zs_v, sem) = refs
    else:
        (x_hbm, src_hbm, dst_hbm, out_hbm,
         acc_sh, src_v, dst_v, rows_v, sem) = refs

    cid = lax.axis_index("c")
    sid = lax.axis_index("s")
    w = cid * 16 + sid

    zero16 = jnp.zeros((16,), jnp.float32)

    def _zrow(i, c):
        for j in range(8):
            rows_v[i, pl.ds(j * 16, 16)] = zero16
        return c
    lax.fori_loop(0, _C, _zrow, 0)

    base = sid * _RPT
    for r in range(_RPT // _C):
        pltpu.sync_copy(rows_v, acc_sh.at[pl.ds(base + r * _C, _C)])

    if with_counts:
        ones16 = jnp.full((16,), 1.0, jnp.float32)
        def _zcnt(i, c):
            zs_v[pl.ds(i * 16, 16)] = zero16
            return c
        lax.fori_loop(0, _RPT // 16, _zcnt, 0)
        for j in range(_C // 16):
            ones_v[pl.ds(j * 16, 16)] = ones16
        pltpu.sync_copy(zs_v, cnt_sh.at[pl.ds(base, _RPT)])

    plsc.subcore_barrier()

    edge0 = w * _EPT

    def _chunk(g, c):
        off = edge0 + g * _C
        pltpu.sync_copy(src_hbm.at[pl.ds(off, _C)], src_v)
        pltpu.sync_copy(dst_hbm.at[pl.ds(off, _C)], dst_v)
        pltpu.async_copy(x_hbm.at[src_v], rows_v, sem).wait()
        if with_counts:
            pltpu.sync_copy(ones_v, cnt_sh.at[dst_v], add=True)
        pltpu.sync_copy(rows_v, acc_sh.at[dst_v], add=True)
        return c

    lax.fori_loop(0, _NCHUNK, _chunk, 0)

    plsc.subcore_barrier()

    out_base = cid * _NP + base
    for r in range(_RPT // _C):
        pltpu.sync_copy(acc_sh.at[pl.ds(base + r * _C, _C)],
                        out_hbm.at[pl.ds(out_base + r * _C, _C)])
    if with_counts:
        pltpu.sync_copy(cnt_sh.at[pl.ds(base, _RPT)],
                        cnt_hbm.at[pl.ds(out_base, _RPT)])


def _make_sc_agg(with_counts):
    mesh = plsc.VectorSubcoreMesh(core_axis_name="c", subcore_axis_name="s")
    out_type = [jax.ShapeDtypeStruct((2 * _NP, _D), jnp.float32)]
    scratch = [
        pltpu.VMEM_SHARED((_NP, _D), jnp.float32),
        pltpu.VMEM((_C,), jnp.int32),
        pltpu.VMEM((_C,), jnp.int32),
        pltpu.VMEM((_C, _D), jnp.float32),
        pltpu.SemaphoreType.DMA,
    ]
    if with_counts:
        out_type.append(jax.ShapeDtypeStruct((2 * _NP,), jnp.float32))
        scratch.insert(1, pltpu.VMEM_SHARED((_NP,), jnp.float32))
        scratch.insert(5, pltpu.VMEM((_C,), jnp.float32))
        scratch.insert(6, pltpu.VMEM((_RPT,), jnp.float32))
    return pl.kernel(
        functools.partial(_sc_agg_body, with_counts),
        mesh=mesh,
        out_type=out_type,
        scratch_types=scratch,
    )


_sc_agg_counts = _make_sc_agg(True)
_sc_agg_plain = _make_sc_agg(False)


def _dense_body(apply_relu, p0, p1, c0, c1, xr, wl, wr, b, out):
    cnt = c0[...] + c1[...]
    inv = 1.0 / jnp.maximum(cnt, 1.0)
    agg = (p0[...] + p1[...]) * inv
    acc = (jnp.dot(agg, wl[...], preferred_element_type=jnp.float32)
           + jnp.dot(xr[...], wr[...], preferred_element_type=jnp.float32)
           + b[...])
    if apply_relu:
        acc = jnp.maximum(acc, 0.0)
    out[...] = acc


_BLK = 1024


def _make_dense(apply_relu):
    row = pl.BlockSpec((_BLK, _D), lambda i: (i, 0))
    col = pl.BlockSpec((_BLK, 1), lambda i: (i, 0))
    full = pl.BlockSpec((_D, _D), lambda i: (0, 0))
    bias = pl.BlockSpec((1, _D), lambda i: (0, 0))
    return pl.pallas_call(
        functools.partial(_dense_body, apply_relu),
        grid=(_NP // _BLK,),
        in_specs=[row, row, col, col, row, full, full, bias],
        out_specs=row,
        out_shape=jax.ShapeDtypeStruct((_NP, _D), jnp.float32),
    )


_dense_relu = _make_dense(True)
_dense_lin = _make_dense(False)


def kernel(x, edge_index, W1_l, W1_r, b1, W2_l, W2_r, b2):
    pad_e = _EP - _E
    src = jnp.concatenate([edge_index[0], jnp.zeros((pad_e,), jnp.int32)])
    dst = jnp.concatenate([edge_index[1],
                           jnp.full((pad_e,), _N, jnp.int32)])
    x_p = jnp.concatenate(
        [x, jnp.zeros((_NP - _N, _D), jnp.float32)], axis=0)
    b1_r = b1.reshape(1, _D)
    b2_r = b2.reshape(1, _D)

    sums1, cnts = _sc_agg_counts(x_p, src, dst)
    p0, p1 = sums1[:_NP], sums1[_NP:]
    c0 = cnts[:_NP].reshape(_NP, 1)
    c1 = cnts[_NP:].reshape(_NP, 1)

    h = _dense_relu(p0, p1, c0, c1, x_p, W1_l, W1_r, b1_r)

    (sums2,) = _sc_agg_plain(h, src, dst)
    q0, q1 = sums2[:_NP], sums2[_NP:]

    out = _dense_lin(q0, q1, c0, c1, h, W2_l, W2_r, b2_r)
    return out[:_N]
